# baseline (speedup 1.0000x reference)
import jax
import jax.numpy as jnp
from jax import lax
from jax.experimental import pallas as pl
from jax.experimental.pallas import tpu as pltpu
from jax.experimental.pallas import tpu_sc as plsc

NC = 2
NS = 16
NW = NC * NS
L = 16

BE = 4096
G = 128
RB = 64

_NEG = -3.4e38


def _tc_precompute(xp, W, b2):
    n_pad, d = xp.shape

    def body(x_ref, w_ref, b_ref, p_ref, q_ref):
        xv = x_ref[...]
        w1 = w_ref[:, :d]
        w2 = w_ref[:, d:]
        dn = (((1,), (1,)), ((), ()))
        q = lax.dot_general(xv, w2, dn, precision=lax.Precision.HIGHEST,
                            preferred_element_type=jnp.float32)
        p = lax.dot_general(xv, w1 - w2, dn, precision=lax.Precision.HIGHEST,
                            preferred_element_type=jnp.float32)
        p_ref[...] = p + b_ref[...]
        q_ref[...] = q

    return pl.pallas_call(
        body,
        out_shape=[
            jax.ShapeDtypeStruct((n_pad, d), jnp.float32),
            jax.ShapeDtypeStruct((n_pad, d), jnp.float32),
        ],
    )(xp, W, b2)


def _sc_segment_max(q, p, dstp, srcp, nv, d, nb):
    n_pad = q.shape[0]
    mesh = plsc.VectorSubcoreMesh(core_axis_name="c", subcore_axis_name="s")

    def body(q_hbm, p_hbm, dst_hbm, src_hbm, out_hbm,
             acc, dstv, srcv, selsrc, selbase, rows, sem):
        iota = lax.iota(jnp.int32, L)
        cid = lax.axis_index("c")
        sid = lax.axis_index("s")
        wid = sid * NC + cid
        lo = wid * nv

        neg = jnp.full((L,), _NEG, jnp.float32)
        zero = jnp.zeros((L,), jnp.int32)

        @pl.loop(0, nv * d, step=L)
        def _(i):
            acc[pl.ds(i, L)] = neg

        @pl.loop(0, BE, step=L)
        def _(i):
            selsrc[pl.ds(i, L)] = zero

        @pl.loop(0, nb)
        def _(blk):
            eb = blk * BE
            pltpu.sync_copy(dst_hbm.at[pl.ds(eb, BE)], dstv)
            pltpu.sync_copy(src_hbm.at[pl.ds(eb, BE)], srcv)

            def scan_body(g, k):
                d16 = dstv[pl.ds(g * L, L)]
                s16 = srcv[pl.ds(g * L, L)]
                m = (d16 >= lo) & (d16 < lo + nv)
                mi = m.astype(jnp.int32)
                cs = plsc.cumsum(mi)
                pos = k + cs - mi
                plsc.store_scatter(selsrc, [pos], s16, mask=m)
                plsc.store_scatter(selbase, [pos], (d16 - lo) * d, mask=m)
                return k + jnp.max(cs)

            k = lax.fori_loop(0, BE // L, scan_body, jnp.int32(0))

            for c in range(BE // G):
                cb = c * G

                @pl.when(k > cb)
                def _():
                    pltpu.async_copy(q_hbm.at[selsrc.at[pl.ds(cb, G)]],
                                     rows, sem).wait()
                    nr = jnp.minimum(k - cb, G)

                    def edge_body(r, carry):
                        base = plsc.load_gather(selbase, [zero + (cb + r)])
                        for f in range(d // L):
                            addr = base + (f * L) + iota
                            av = plsc.load_gather(acc, [addr])
                            rv = rows[r, pl.ds(f * L, L)]
                            plsc.store_scatter(acc, [addr],
                                               jnp.maximum(av, rv))
                        return carry

                    lax.fori_loop(0, nr, edge_body, jnp.int32(0))

        for off in range(0, nv, RB):
            pltpu.sync_copy(p_hbm.at[pl.ds(lo + off, RB)],
                            rows.at[pl.ds(0, RB)])

            @pl.loop(0, RB)
            def _(i):
                for f in range(d // L):
                    pv = rows[i, pl.ds(f * L, L)]
                    av = acc[pl.ds((off + i) * d + f * L, L)]
                    rows[i, pl.ds(f * L, L)] = jnp.maximum(pv + av, 0.0)

            pltpu.sync_copy(rows.at[pl.ds(0, RB)],
                            out_hbm.at[pl.ds(lo + off, RB)])

    kern = pl.kernel(
        body,
        out_type=jax.ShapeDtypeStruct((n_pad, d), jnp.float32),
        mesh=mesh,
        compiler_params=pltpu.CompilerParams(needs_layout_passes=False),
        scratch_types=[
            pltpu.VMEM((nv * d,), jnp.float32),
            pltpu.VMEM((BE,), jnp.int32),
            pltpu.VMEM((BE,), jnp.int32),
            pltpu.VMEM((BE,), jnp.int32),
            pltpu.VMEM((BE,), jnp.int32),
            pltpu.VMEM((G, d), jnp.float32),
            pltpu.SemaphoreType.DMA,
        ],
    )
    return kern(q, p, dstp, srcp)


def kernel(x, edge_index, W, b):
    n, d = x.shape
    e = edge_index.shape[1]

    nv = -(-n // NW)
    nv = -(-nv // 8) * 8
    n_pad = NW * nv
    nb = -(-e // BE)
    e_pad = nb * BE

    xp = jnp.pad(x, ((0, n_pad - n), (0, 0)))
    srcp = jnp.pad(edge_index[0], (0, e_pad - e))
    dstp = jnp.pad(edge_index[1], (0, e_pad - e), constant_values=-1)

    p, q = _tc_precompute(xp, W, b.reshape(1, d))
    outp = _sc_segment_max(q, p, dstp, srcp, nv, d, nb)
    return outp[:n]
